# baseline (speedup 1.0000x reference)
import jax
import jax.numpy as jnp
from jax import lax
from jax.experimental import pallas as pl
from jax.experimental.pallas import tpu as pltpu

_CHUNK = 16384
_NCORES = 2


def _round_up(n, m):
    return (n + m - 1) // m * m


def _tmm_kernel(xt_ref, w_ref, bt_ref, ot_ref):
    acc = lax.dot_general(
        w_ref[...],
        xt_ref[...],
        dimension_numbers=(((1,), (0,)), ((), ())),
        preferred_element_type=jnp.float32,
    )
    ot_ref[...] = (acc + bt_ref[...]).astype(ot_ref.dtype)


@jax.jit
def _forward(x, weight, bias2d):
    B, d_in = x.shape
    d_out = weight.shape[0]

    B_p = _round_up(B, _NCORES * _CHUNK)
    if B_p != B:
        x = jnp.pad(x, ((0, B_p - B), (0, 0)))
    steps = B_p // (_NCORES * _CHUNK)

    xt = x.T
    bt = bias2d.T

    out_t = pl.pallas_call(
        _tmm_kernel,
        grid=(_NCORES, steps),
        in_specs=[
            pl.BlockSpec((d_in, _CHUNK), lambda c, j: (0, c * steps + j)),
            pl.BlockSpec((d_out, d_in), lambda c, j: (0, 0)),
            pl.BlockSpec((d_out, 1), lambda c, j: (0, 0)),
        ],
        out_specs=pl.BlockSpec((d_out, _CHUNK), lambda c, j: (0, c * steps + j)),
        out_shape=jax.ShapeDtypeStruct((d_out, B_p), x.dtype),
        compiler_params=pltpu.CompilerParams(
            dimension_semantics=("parallel", "arbitrary"),
        ),
        cost_estimate=pl.CostEstimate(
            flops=2 * B_p * d_in * d_out,
            bytes_accessed=B_p * (d_in + d_out) * 4,
            transcendentals=0,
        ),
    )(xt, weight, bt)

    out = out_t.T
    if B_p != B:
        out = out[:B]
    return out


def kernel(x, weight, bias2d):
    return _forward(x, weight, bias2d)


# speedup vs baseline: 1.2330x; 1.2330x over previous
import jax
import jax.numpy as jnp
from jax import lax
from jax.experimental import pallas as pl
from jax.experimental.pallas import tpu as pltpu

_CHUNK = 65536
_NCORES = 2


def _round_up(n, m):
    return (n + m - 1) // m * m


def _tmm_kernel(xt_ref, w_ref, bt_ref, ot_ref):
    acc = lax.dot_general(
        w_ref[...],
        xt_ref[...],
        dimension_numbers=(((1,), (0,)), ((), ())),
        preferred_element_type=jnp.float32,
    )
    ot_ref[...] = (acc + bt_ref[...]).astype(ot_ref.dtype)


@jax.jit
def _forward(x, weight, bias2d):
    B, d_in = x.shape
    d_out = weight.shape[0]

    B_p = _round_up(B, _NCORES * _CHUNK)
    if B_p != B:
        x = jnp.pad(x, ((0, B_p - B), (0, 0)))
    steps = B_p // (_NCORES * _CHUNK)

    xt = x.T
    bt = bias2d.T

    out_t = pl.pallas_call(
        _tmm_kernel,
        grid=(_NCORES, steps),
        in_specs=[
            pl.BlockSpec((d_in, _CHUNK), lambda c, j: (0, c * steps + j)),
            pl.BlockSpec((d_out, d_in), lambda c, j: (0, 0)),
            pl.BlockSpec((d_out, 1), lambda c, j: (0, 0)),
        ],
        out_specs=pl.BlockSpec((d_out, _CHUNK), lambda c, j: (0, c * steps + j)),
        out_shape=jax.ShapeDtypeStruct((d_out, B_p), x.dtype),
        compiler_params=pltpu.CompilerParams(
            dimension_semantics=("parallel", "arbitrary"),
        ),
        cost_estimate=pl.CostEstimate(
            flops=2 * B_p * d_in * d_out,
            bytes_accessed=B_p * (d_in + d_out) * 4,
            transcendentals=0,
        ),
    )(xt, weight, bt)

    out = out_t.T
    if B_p != B:
        out = out[:B]
    return out


def kernel(x, weight, bias2d):
    return _forward(x, weight, bias2d)
